# baseline (speedup 1.0000x reference)
import functools

import jax
import jax.numpy as jnp
from jax import lax
from jax.experimental import pallas as pl
from jax.experimental.pallas import tpu as pltpu
from jax.experimental.pallas import tpu_sc as plsc

N = 10000
E = 320000
D = 128
H = 256
O = 128
HH = H // 2

NC = 2
NS = 16
LANES = 16

NPAD = 10240
RPT = NPAD // NS
ZROWS = 40

EPS = 20480
EP = EPS * NS
SEG_C = 80
SEG_NCHUNK = EPS // SEG_C
SEG_NBUF = 2
SEG_NOUTER = SEG_NCHUNK // SEG_NBUF

DEG_E = E // (NC * NS)
DEG_C = 80
DEG_NCHUNK = DEG_E // DEG_C
DEG_NBUF = 5
DEG_NOUTER = DEG_NCHUNK // DEG_NBUF

_mesh = plsc.VectorSubcoreMesh(core_axis_name="c", subcore_axis_name="s")


def _zero_vmem_1d(ref, nwords):
    def body(i, _):
        ref[pl.ds(i * LANES, LANES)] = jnp.zeros((LANES,), jnp.float32)
        return 0
    lax.fori_loop(0, nwords // LANES, body, 0)


def _deg_body(dst_hbm, out_hbm, deg_sh,
              db0, db1, db2, db3, db4, ones, rbuf,
              ds0, ds1, ds2, ds3, ds4):
    c = lax.axis_index("c")
    s = lax.axis_index("s")
    tid = c * NS + s
    dibs = (db0, db1, db2, db3, db4)
    sems = (ds0, ds1, ds2, ds3, ds4)

    seg = NPAD // NS
    r0 = s * seg
    _zero_vmem_1d(rbuf, seg)
    pltpu.sync_copy(rbuf, deg_sh.at[pl.ds(r0, seg)])

    def fill_ones(i, _):
        ones[pl.ds(i * LANES, LANES)] = jnp.ones((LANES,), jnp.float32)
        return 0
    lax.fori_loop(0, DEG_C // LANES, fill_ones, 0)
    plsc.subcore_barrier()

    base = tid * DEG_E

    def load_idx(k, b):
        pltpu.async_copy(dst_hbm.at[pl.ds(base + k * DEG_C, DEG_C)],
                         dibs[b], sems[b])

    for b in range(DEG_NBUF):
        load_idx(b, b)

    def outer(ko, _):
        for b in range(DEG_NBUF):
            k = ko * DEG_NBUF + b
            pltpu.make_async_copy(
                dst_hbm.at[pl.ds(base + k * DEG_C, DEG_C)], dibs[b], sems[b]
            ).wait()
            pltpu.sync_copy(ones, deg_sh.at[dibs[b]], add=True)
            kp = k + DEG_NBUF

            @pl.when(kp < DEG_NCHUNK)
            def _():
                load_idx(kp, b)
        return 0

    lax.fori_loop(0, DEG_NOUTER, outer, 0)
    plsc.subcore_barrier()
    pltpu.sync_copy(deg_sh.at[pl.ds(r0, seg)], out_hbm.at[c, pl.ds(r0, seg)])


_deg_call = pl.kernel(
    _deg_body,
    out_type=jax.ShapeDtypeStruct((NC, NPAD), jnp.float32),
    mesh=_mesh,
    scratch_types=(
        [pltpu.VMEM_SHARED((NPAD,), jnp.float32)]
        + [pltpu.VMEM((DEG_C,), jnp.int32) for _ in range(DEG_NBUF)]
        + [pltpu.VMEM((DEG_C,), jnp.float32),
           pltpu.VMEM((NPAD // NS,), jnp.float32)]
        + [pltpu.SemaphoreType.DMA for _ in range(DEG_NBUF)]
    ),
)


def _seg_body(hp_hbm, src_hbm, dst_hbm, z_hbm, out_hbm, acc,
              sib0, sib1, dib0, dib1, rb0, rb1, gs0, gs1):
    c = lax.axis_index("c")
    s = lax.axis_index("s")
    sibs = (sib0, sib1)
    dibs = (dib0, dib1)
    rbs = (rb0, rb1)
    gsems = (gs0, gs1)

    r0 = s * RPT
    for j in range(RPT // ZROWS):
        pltpu.sync_copy(z_hbm, acc.at[pl.ds(r0 + j * ZROWS, ZROWS)])
    plsc.subcore_barrier()

    base = s * EPS
    off = c * N

    def load_and_gather(k, b):
        pltpu.sync_copy(src_hbm.at[pl.ds(base + k * SEG_C, SEG_C)], sibs[b])
        for i in range(SEG_C // LANES):
            sl = pl.ds(i * LANES, LANES)
            sibs[b][sl] = sibs[b][sl] + off
        pltpu.async_copy(hp_hbm.at[sibs[b]], rbs[b], gsems[b])
        pltpu.sync_copy(dst_hbm.at[pl.ds(base + k * SEG_C, SEG_C)], dibs[b])

    for b in range(SEG_NBUF):
        load_and_gather(b, b)

    def outer(ko, _):
        for b in range(SEG_NBUF):
            k = ko * SEG_NBUF + b
            pltpu.make_async_copy(hp_hbm.at[sibs[b]], rbs[b], gsems[b]).wait()
            pltpu.sync_copy(rbs[b], acc.at[dibs[b]], add=True)
            kp = k + SEG_NBUF

            @pl.when(kp < SEG_NCHUNK)
            def _():
                load_and_gather(kp, b)
        return 0

    lax.fori_loop(0, SEG_NOUTER, outer, 0)
    plsc.subcore_barrier()
    pltpu.sync_copy(acc.at[pl.ds(r0, RPT)],
                    out_hbm.at[pl.ds(c * NPAD + r0, RPT)])


_seg_call = pl.kernel(
    _seg_body,
    out_type=jax.ShapeDtypeStruct((2 * NPAD, HH), jnp.float32),
    mesh=_mesh,
    scratch_types=(
        [pltpu.VMEM_SHARED((NPAD, HH), jnp.float32)]
        + [pltpu.VMEM((SEG_C,), jnp.int32) for _ in range(SEG_NBUF)]
        + [pltpu.VMEM((SEG_C,), jnp.int32) for _ in range(SEG_NBUF)]
        + [pltpu.VMEM((SEG_C, HH), jnp.float32) for _ in range(SEG_NBUF)]
        + [pltpu.SemaphoreType.DMA for _ in range(SEG_NBUF)]
    ),
)


BN = 1000
GRID = N // BN


def _dis_body(degp_ref, out_ref):
    deg = degp_ref[0] + degp_ref[1] + 1.0
    out_ref[...] = lax.rsqrt(deg)


def _dis_call(degp3):
    return pl.pallas_call(
        _dis_body,
        out_shape=jax.ShapeDtypeStruct((NPAD // 128, 128), jnp.float32),
    )(degp3)


def _tc1_body(x_ref, w1_ref, disb_ref, out_ref):
    xw = jnp.dot(x_ref[...], w1_ref[...], preferred_element_type=jnp.float32)
    d = disb_ref[...]
    out_ref[0] = d * xw[:, :HH]
    out_ref[1] = d * xw[:, HH:]


def _tc1_call(x, W1, disb):
    return pl.pallas_call(
        _tc1_body,
        grid=(GRID,),
        in_specs=[
            pl.BlockSpec((BN, D), lambda i: (i, 0)),
            pl.BlockSpec((D, H), lambda i: (0, 0)),
            pl.BlockSpec((BN, HH), lambda i: (i, 0)),
        ],
        out_specs=pl.BlockSpec((2, BN, HH), lambda i: (0, i, 0)),
        out_shape=jax.ShapeDtypeStruct((2, N, HH), jnp.float32),
    )(x, W1, disb)


def _tc2_body(a_ref, hp_ref, disb_ref, b1_ref, w2_ref, out_ref):
    d = disb_ref[...]
    b1 = b1_ref[...]
    h1l = jnp.maximum(d * (a_ref[0] + hp_ref[0]) + b1[0], 0.0)
    h1r = jnp.maximum(d * (a_ref[1] + hp_ref[1]) + b1[1], 0.0)
    w2 = w2_ref[...]
    g = (jnp.dot(h1l, w2[:HH, :], preferred_element_type=jnp.float32)
         + jnp.dot(h1r, w2[HH:, :], preferred_element_type=jnp.float32))
    out_ref[0] = d * g[:, :HH]
    out_ref[1] = d * g[:, HH:]


def _tc2_call(A1, hp, disb, b1r, W2):
    return pl.pallas_call(
        _tc2_body,
        grid=(GRID,),
        in_specs=[
            pl.BlockSpec((2, BN, HH), lambda i: (0, i, 0)),
            pl.BlockSpec((2, BN, HH), lambda i: (0, i, 0)),
            pl.BlockSpec((BN, HH), lambda i: (i, 0)),
            pl.BlockSpec((2, HH), lambda i: (0, 0)),
            pl.BlockSpec((H, H), lambda i: (0, 0)),
        ],
        out_specs=pl.BlockSpec((2, BN, HH), lambda i: (0, i, 0)),
        out_shape=jax.ShapeDtypeStruct((2, N, HH), jnp.float32),
    )(A1, hp, disb, b1r, W2)


def _tc3_body(a_ref, gp_ref, disb_ref, b2_ref, wf_ref, bf_ref, out_ref):
    d = disb_ref[...]
    b2 = b2_ref[...]
    h2l = jnp.maximum(d * (a_ref[0] + gp_ref[0]) + b2[0], 0.0)
    h2r = jnp.maximum(d * (a_ref[1] + gp_ref[1]) + b2[1], 0.0)
    wf = wf_ref[...]
    out_ref[...] = (jnp.dot(h2l, wf[:HH, :], preferred_element_type=jnp.float32)
                    + jnp.dot(h2r, wf[HH:, :], preferred_element_type=jnp.float32)
                    + bf_ref[...])


def _tc3_call(A2, gp, disb, b2r, Wf, bfr):
    return pl.pallas_call(
        _tc3_body,
        grid=(GRID,),
        in_specs=[
            pl.BlockSpec((2, BN, HH), lambda i: (0, i, 0)),
            pl.BlockSpec((2, BN, HH), lambda i: (0, i, 0)),
            pl.BlockSpec((BN, HH), lambda i: (i, 0)),
            pl.BlockSpec((2, HH), lambda i: (0, 0)),
            pl.BlockSpec((H, O), lambda i: (0, 0)),
            pl.BlockSpec((1, O), lambda i: (0, 0)),
        ],
        out_specs=pl.BlockSpec((BN, O), lambda i: (i, 0)),
        out_shape=jax.ShapeDtypeStruct((N, O), jnp.float32),
    )(A2, gp, disb, b2r, Wf, bfr)


def kernel(x, edge_index, edge_attr, W1, b1, W2, b2, We, be, Wf, bf):
    del edge_attr, We, be
    src = edge_index[0]
    dst = edge_index[1]

    degp = _deg_call(dst)
    disb2d = _dis_call(degp.reshape(NC, NPAD // 128, 128))
    dis = disb2d.reshape(NPAD)[:N]
    disb = jnp.broadcast_to(dis[:, None], (N, HH))

    zrows = jnp.zeros((ZROWS, HH), jnp.float32)

    srcp = jnp.concatenate([src, jnp.zeros((EP - E,), jnp.int32)])
    pad_dst = N + jnp.arange(EP - E, dtype=jnp.int32) % (NPAD - N)
    dstp = jnp.concatenate([dst, pad_dst])

    hp = _tc1_call(x, W1, disb)
    a1 = _seg_call(hp.reshape(2 * N, HH), srcp, dstp,
                   zrows).reshape(2, NPAD, HH)
    gp = _tc2_call(a1, hp, disb, b1.reshape(NC, HH), W2)
    a2 = _seg_call(gp.reshape(2 * N, HH), srcp, dstp,
                   zrows).reshape(2, NPAD, HH)
    return _tc3_call(a2, gp, disb, b2.reshape(NC, HH), Wf, bf.reshape(1, O))


# speedup vs baseline: 1.6564x; 1.6564x over previous
import functools

import jax
import jax.numpy as jnp
from jax import lax
from jax.experimental import pallas as pl
from jax.experimental.pallas import tpu as pltpu
from jax.experimental.pallas import tpu_sc as plsc

N = 10000
E = 320000
D = 128
H = 256
O = 128
HH = H // 2

NC = 2
NS = 16
LANES = 16

NPAD = 10240
RPT = NPAD // NS
ZROWS = 40

EPS = 20480
EP = EPS * NS
SEG_C = 80
SEG_NCHUNK = EPS // SEG_C
SEG_NBUF = 2
SEG_NOUTER = SEG_NCHUNK // SEG_NBUF

DEG_E = E // (NC * NS)
DEG_C = 80
DEG_NCHUNK = DEG_E // DEG_C
DEG_NBUF = 5
DEG_NOUTER = DEG_NCHUNK // DEG_NBUF

_mesh = plsc.VectorSubcoreMesh(core_axis_name="c", subcore_axis_name="s")


def _zero_vmem_1d(ref, nwords):
    def body(i, _):
        ref[pl.ds(i * LANES, LANES)] = jnp.zeros((LANES,), jnp.float32)
        return 0
    lax.fori_loop(0, nwords // LANES, body, 0)


def _deg_body(dst_hbm, out_hbm, deg_sh,
              db0, db1, db2, db3, db4, ones, rbuf,
              ds0, ds1, ds2, ds3, ds4):
    c = lax.axis_index("c")
    s = lax.axis_index("s")
    tid = c * NS + s
    dibs = (db0, db1, db2, db3, db4)
    sems = (ds0, ds1, ds2, ds3, ds4)

    seg = NPAD // NS
    r0 = s * seg
    _zero_vmem_1d(rbuf, seg)
    pltpu.sync_copy(rbuf, deg_sh.at[pl.ds(r0, seg)])

    def fill_ones(i, _):
        ones[pl.ds(i * LANES, LANES)] = jnp.ones((LANES,), jnp.float32)
        return 0
    lax.fori_loop(0, DEG_C // LANES, fill_ones, 0)
    plsc.subcore_barrier()

    base = tid * DEG_E

    def load_idx(k, b):
        pltpu.async_copy(dst_hbm.at[pl.ds(base + k * DEG_C, DEG_C)],
                         dibs[b], sems[b])

    for b in range(DEG_NBUF):
        load_idx(b, b)

    def outer(ko, _):
        for b in range(DEG_NBUF):
            k = ko * DEG_NBUF + b
            pltpu.make_async_copy(
                dst_hbm.at[pl.ds(base + k * DEG_C, DEG_C)], dibs[b], sems[b]
            ).wait()
            pltpu.sync_copy(ones, deg_sh.at[dibs[b]], add=True)
            kp = k + DEG_NBUF

            @pl.when(kp < DEG_NCHUNK)
            def _():
                load_idx(kp, b)
        return 0

    lax.fori_loop(0, DEG_NOUTER, outer, 0)
    plsc.subcore_barrier()
    pltpu.sync_copy(deg_sh.at[pl.ds(r0, seg)], out_hbm.at[c, pl.ds(r0, seg)])


_deg_call = pl.kernel(
    _deg_body,
    out_type=jax.ShapeDtypeStruct((NC, NPAD), jnp.float32),
    mesh=_mesh,
    scratch_types=(
        [pltpu.VMEM_SHARED((NPAD,), jnp.float32)]
        + [pltpu.VMEM((DEG_C,), jnp.int32) for _ in range(DEG_NBUF)]
        + [pltpu.VMEM((DEG_C,), jnp.float32),
           pltpu.VMEM((NPAD // NS,), jnp.float32)]
        + [pltpu.SemaphoreType.DMA for _ in range(DEG_NBUF)]
    ),
)


def _seg_body(hp_hbm, src_hbm, dst_hbm, z_hbm, out_hbm, acc,
              sib0, sib1, dib0, dib1, rb0, rb1, gs0, gs1):
    c = lax.axis_index("c")
    s = lax.axis_index("s")
    sibs = (sib0, sib1)
    dibs = (dib0, dib1)
    rbs = (rb0, rb1)
    gsems = (gs0, gs1)

    r0 = s * RPT
    for j in range(RPT // ZROWS):
        pltpu.sync_copy(z_hbm, acc.at[pl.ds(r0 + j * ZROWS, ZROWS)])
    plsc.subcore_barrier()

    base = s * EPS
    off = c * N

    def load_and_gather(k, b):
        pltpu.sync_copy(src_hbm.at[pl.ds(base + k * SEG_C, SEG_C)], sibs[b])
        for i in range(SEG_C // LANES):
            sl = pl.ds(i * LANES, LANES)
            sibs[b][sl] = sibs[b][sl] + off
        pltpu.async_copy(hp_hbm.at[sibs[b]], rbs[b], gsems[b])
        pltpu.sync_copy(dst_hbm.at[pl.ds(base + k * SEG_C, SEG_C)], dibs[b])

    for b in range(SEG_NBUF):
        load_and_gather(b, b)

    def outer(ko, _):
        for b in range(SEG_NBUF):
            k = ko * SEG_NBUF + b
            pltpu.make_async_copy(hp_hbm.at[sibs[b]], rbs[b], gsems[b]).wait()
            pltpu.sync_copy(rbs[b], acc.at[dibs[b]], add=True)
            kp = k + SEG_NBUF

            @pl.when(kp < SEG_NCHUNK)
            def _():
                load_and_gather(kp, b)
        return 0

    lax.fori_loop(0, SEG_NOUTER, outer, 0)
    plsc.subcore_barrier()
    pltpu.sync_copy(acc.at[pl.ds(r0, RPT)],
                    out_hbm.at[pl.ds(c * NPAD + r0, RPT)])


_seg_call = pl.kernel(
    _seg_body,
    out_type=jax.ShapeDtypeStruct((2 * NPAD, HH), jnp.float32),
    mesh=_mesh,
    scratch_types=(
        [pltpu.VMEM_SHARED((NPAD, HH), jnp.float32)]
        + [pltpu.VMEM((SEG_C,), jnp.int32) for _ in range(SEG_NBUF)]
        + [pltpu.VMEM((SEG_C,), jnp.int32) for _ in range(SEG_NBUF)]
        + [pltpu.VMEM((SEG_C, HH), jnp.float32) for _ in range(SEG_NBUF)]
        + [pltpu.SemaphoreType.DMA for _ in range(SEG_NBUF)]
    ),
)


BN = 1000
GRID = N // BN


def _dis_body(degp_ref, out_ref):
    deg = degp_ref[0] + degp_ref[1] + 1.0
    out_ref[...] = lax.rsqrt(deg)


def _dis_call(degp3):
    return pl.pallas_call(
        _dis_body,
        out_shape=jax.ShapeDtypeStruct((NPAD // 128, 128), jnp.float32),
    )(degp3)


def _tc1_body(x_ref, w1_ref, disb_ref, out_ref):
    xw = jnp.dot(x_ref[...], w1_ref[...], preferred_element_type=jnp.float32)
    d = disb_ref[...]
    out_ref[0] = d * xw[:, :HH]
    out_ref[1] = d * xw[:, HH:]


def _tc1_call(x, W1, disb):
    return pl.pallas_call(
        _tc1_body,
        grid=(GRID,),
        in_specs=[
            pl.BlockSpec((BN, D), lambda i: (i, 0)),
            pl.BlockSpec((D, H), lambda i: (0, 0)),
            pl.BlockSpec((BN, HH), lambda i: (i, 0)),
        ],
        out_specs=pl.BlockSpec((2, BN, HH), lambda i: (0, i, 0)),
        out_shape=jax.ShapeDtypeStruct((2, N, HH), jnp.float32),
    )(x, W1, disb)


def _tc2_body(a_ref, hp_ref, disb_ref, b1_ref, w2_ref, out_ref):
    d = disb_ref[...]
    b1 = b1_ref[...]
    h1l = jnp.maximum(d * (a_ref[0] + hp_ref[0]) + b1[0], 0.0)
    h1r = jnp.maximum(d * (a_ref[1] + hp_ref[1]) + b1[1], 0.0)
    w2 = w2_ref[...]
    g = (jnp.dot(h1l, w2[:HH, :], preferred_element_type=jnp.float32)
         + jnp.dot(h1r, w2[HH:, :], preferred_element_type=jnp.float32))
    out_ref[0] = d * g[:, :HH]
    out_ref[1] = d * g[:, HH:]


def _tc2_call(A1, hp, disb, b1r, W2):
    return pl.pallas_call(
        _tc2_body,
        grid=(GRID,),
        in_specs=[
            pl.BlockSpec((2, BN, HH), lambda i: (0, i, 0)),
            pl.BlockSpec((2, BN, HH), lambda i: (0, i, 0)),
            pl.BlockSpec((BN, HH), lambda i: (i, 0)),
            pl.BlockSpec((2, HH), lambda i: (0, 0)),
            pl.BlockSpec((H, H), lambda i: (0, 0)),
        ],
        out_specs=pl.BlockSpec((2, BN, HH), lambda i: (0, i, 0)),
        out_shape=jax.ShapeDtypeStruct((2, N, HH), jnp.float32),
    )(A1, hp, disb, b1r, W2)


def _tc3_body(a_ref, gp_ref, disb_ref, b2_ref, wf_ref, bf_ref, out_ref):
    d = disb_ref[...]
    b2 = b2_ref[...]
    h2l = jnp.maximum(d * (a_ref[0] + gp_ref[0]) + b2[0], 0.0)
    h2r = jnp.maximum(d * (a_ref[1] + gp_ref[1]) + b2[1], 0.0)
    wf = wf_ref[...]
    out_ref[...] = (jnp.dot(h2l, wf[:HH, :], preferred_element_type=jnp.float32)
                    + jnp.dot(h2r, wf[HH:, :], preferred_element_type=jnp.float32)
                    + bf_ref[...])


def _tc3_call(A2, gp, disb, b2r, Wf, bfr):
    return pl.pallas_call(
        _tc3_body,
        grid=(GRID,),
        in_specs=[
            pl.BlockSpec((2, BN, HH), lambda i: (0, i, 0)),
            pl.BlockSpec((2, BN, HH), lambda i: (0, i, 0)),
            pl.BlockSpec((BN, HH), lambda i: (i, 0)),
            pl.BlockSpec((2, HH), lambda i: (0, 0)),
            pl.BlockSpec((H, O), lambda i: (0, 0)),
            pl.BlockSpec((1, O), lambda i: (0, 0)),
        ],
        out_specs=pl.BlockSpec((BN, O), lambda i: (i, 0)),
        out_shape=jax.ShapeDtypeStruct((N, O), jnp.float32),
    )(A2, gp, disb, b2r, Wf, bfr)


def kernel(x, edge_index, edge_attr, W1, b1, W2, b2, We, be, Wf, bf):
    del edge_attr, We, be
    src = edge_index[0]
    dst = edge_index[1]

    degp = _deg_call(dst)
    disb2d = _dis_call(degp.reshape(NC, NPAD // 128, 128))
    dis = disb2d.reshape(NPAD)[:N]
    disb = jnp.broadcast_to(dis[:, None], (N, HH))

    zrows = jnp.zeros((ZROWS, HH), jnp.float32)

    pad_iota = jnp.arange(EP - E, dtype=jnp.int32)
    srcp = jnp.concatenate([src, pad_iota % N])
    dstp = jnp.concatenate([dst, N + pad_iota % (NPAD - N)])

    hp = _tc1_call(x, W1, disb)
    a1 = _seg_call(hp.reshape(2 * N, HH), srcp, dstp,
                   zrows).reshape(2, NPAD, HH)
    gp = _tc2_call(a1, hp, disb, b1.reshape(NC, HH), W2)
    a2 = _seg_call(gp.reshape(2 * N, HH), srcp, dstp,
                   zrows).reshape(2, NPAD, HH)
    return _tc3_call(a2, gp, disb, b2.reshape(NC, HH), Wf, bf.reshape(1, O))


# speedup vs baseline: 2.6637x; 1.6082x over previous
import functools

import jax
import jax.numpy as jnp
from jax import lax
from jax.experimental import pallas as pl
from jax.experimental.pallas import tpu as pltpu
from jax.experimental.pallas import tpu_sc as plsc

N = 10000
E = 320000
D = 128
H = 256
O = 128
HH = H // 2

NC = 2
NS = 16
LANES = 16

NPAD = 10240
RPT = NPAD // NS

SEG_C = 80
SEG_IBUF = 4
EPT = 20480
EP = EPT * NS
SEG_NCHUNK = EPT // SEG_C

DEG_E = E // (NC * NS)
DEG_C = 80
DEG_NCHUNK = DEG_E // DEG_C
DEG_NBUF = 5
DEG_NOUTER = DEG_NCHUNK // DEG_NBUF

_mesh = plsc.VectorSubcoreMesh(core_axis_name="c", subcore_axis_name="s")


def _zero_vmem_1d(ref, nwords):
    def body(i, _):
        ref[pl.ds(i * LANES, LANES)] = jnp.zeros((LANES,), jnp.float32)
        return 0
    lax.fori_loop(0, nwords // LANES, body, 0)


def _deg_body(dst_hbm, out_hbm, deg_sh,
              db0, db1, db2, db3, db4, ones, rbuf,
              ds0, ds1, ds2, ds3, ds4):
    c = lax.axis_index("c")
    s = lax.axis_index("s")
    tid = c * NS + s
    dibs = (db0, db1, db2, db3, db4)
    sems = (ds0, ds1, ds2, ds3, ds4)

    seg = NPAD // NS
    r0 = s * seg
    _zero_vmem_1d(rbuf, seg)
    pltpu.sync_copy(rbuf, deg_sh.at[pl.ds(r0, seg)])

    def fill_ones(i, _):
        ones[pl.ds(i * LANES, LANES)] = jnp.ones((LANES,), jnp.float32)
        return 0
    lax.fori_loop(0, DEG_C // LANES, fill_ones, 0)
    plsc.subcore_barrier()

    base = tid * DEG_E

    def load_idx(k, b):
        pltpu.async_copy(dst_hbm.at[pl.ds(base + k * DEG_C, DEG_C)],
                         dibs[b], sems[b])

    for b in range(DEG_NBUF):
        load_idx(b, b)

    def outer(ko, _):
        for b in range(DEG_NBUF):
            k = ko * DEG_NBUF + b
            pltpu.make_async_copy(
                dst_hbm.at[pl.ds(base + k * DEG_C, DEG_C)], dibs[b], sems[b]
            ).wait()
            pltpu.sync_copy(ones, deg_sh.at[dibs[b]], add=True)
            kp = k + DEG_NBUF

            @pl.when(kp < DEG_NCHUNK)
            def _():
                load_idx(kp, b)
        return 0

    lax.fori_loop(0, DEG_NOUTER, outer, 0)
    plsc.subcore_barrier()
    pltpu.sync_copy(deg_sh.at[pl.ds(r0, seg)], out_hbm.at[c, pl.ds(r0, seg)])


_deg_call = pl.kernel(
    _deg_body,
    out_type=jax.ShapeDtypeStruct((NC, NPAD), jnp.float32),
    mesh=_mesh,
    scratch_types=(
        [pltpu.VMEM_SHARED((NPAD,), jnp.float32)]
        + [pltpu.VMEM((DEG_C,), jnp.int32) for _ in range(DEG_NBUF)]
        + [pltpu.VMEM((DEG_C,), jnp.float32),
           pltpu.VMEM((NPAD // NS,), jnp.float32)]
        + [pltpu.SemaphoreType.DMA for _ in range(DEG_NBUF)]
    ),
)


def _seg_body(hp_hbm, src_hbm, dst_hbm, out_hbm, acc,
              sib0, sib1, sib2, sib3, dib0, dib1, dib2, dib3,
              rb0, rb1, gs0, gs1, is0, is1, is2, is3):
    c = lax.axis_index("c")
    s = lax.axis_index("s")
    sibs = (sib0, sib1, sib2, sib3)
    dibs = (dib0, dib1, dib2, dib3)
    rbs = (rb0, rb1)
    gsems = (gs0, gs1)
    isems = (is0, is1, is2, is3)

    def zrb(i, _):
        rb0[i // (HH // LANES),
            pl.ds((i % (HH // LANES)) * LANES, LANES)] = (
                jnp.zeros((LANES,), jnp.float32))
        return 0
    lax.fori_loop(0, SEG_C * HH // LANES, zrb, 0)
    r0 = s * RPT
    for j in range(RPT // SEG_C):
        pltpu.sync_copy(rb0, acc.at[pl.ds(r0 + j * SEG_C, SEG_C)])
    plsc.subcore_barrier()

    base = s * EPT
    off = c * N

    def idx_load(q, p):
        pltpu.async_copy(src_hbm.at[pl.ds(base + q * SEG_C, SEG_C)],
                         sibs[p], isems[p])
        pltpu.async_copy(dst_hbm.at[pl.ds(base + q * SEG_C, SEG_C)],
                         dibs[p], isems[p])

    def idx_wait(p):
        pltpu.make_async_copy(src_hbm.at[pl.ds(0, SEG_C)], sibs[p],
                              isems[p]).wait()
        pltpu.make_async_copy(dst_hbm.at[pl.ds(0, SEG_C)], dibs[p],
                              isems[p]).wait()

    def rebase_and_gather(p, b):
        for i in range(SEG_C // LANES):
            sl = pl.ds(i * LANES, LANES)
            sibs[p][sl] = sibs[p][sl] + off
        pltpu.async_copy(hp_hbm.at[sibs[p]], rbs[b], gsems[b])

    for q in range(SEG_IBUF):
        idx_load(q, q)
    for q in range(2):
        idx_wait(q)
        rebase_and_gather(q, q)

    def outer(ko, _):
        for u in range(SEG_IBUF):
            k = ko * SEG_IBUF + u
            b = u % 2
            pltpu.make_async_copy(hp_hbm.at[sibs[u]], rbs[b],
                                  gsems[b]).wait()
            pltpu.sync_copy(rbs[b], acc.at[dibs[u]], add=True)

            kf = k + SEG_IBUF

            @pl.when(kf < SEG_NCHUNK)
            def _():
                idx_load(kf, u)

            kp = k + 2
            up = (u + 2) % SEG_IBUF

            @pl.when(kp < SEG_NCHUNK)
            def _():
                idx_wait(up)
                rebase_and_gather(up, b)
        return 0

    lax.fori_loop(0, SEG_NCHUNK // SEG_IBUF, outer, 0)
    plsc.subcore_barrier()
    pltpu.sync_copy(acc.at[pl.ds(r0, RPT)],
                    out_hbm.at[pl.ds(c * NPAD + r0, RPT)])


_seg_call = pl.kernel(
    _seg_body,
    out_type=jax.ShapeDtypeStruct((2 * NPAD, HH), jnp.float32),
    mesh=_mesh,
    scratch_types=(
        [pltpu.VMEM_SHARED((NPAD, HH), jnp.float32)]
        + [pltpu.VMEM((SEG_C,), jnp.int32) for _ in range(2 * SEG_IBUF)]
        + [pltpu.VMEM((SEG_C, HH), jnp.float32) for _ in range(2)]
        + [pltpu.SemaphoreType.DMA for _ in range(2 + SEG_IBUF)]
    ),
)


BN = 1000
GRID = N // BN


def _dis_body(degp_ref, out_ref):
    deg = degp_ref[0] + degp_ref[1] + 1.0
    out_ref[...] = lax.rsqrt(deg)


def _dis_call(degp3):
    return pl.pallas_call(
        _dis_body,
        out_shape=jax.ShapeDtypeStruct((NPAD // 128, 128), jnp.float32),
    )(degp3)


def _tc1_body(x_ref, w1_ref, disb_ref, out_ref):
    xw = jnp.dot(x_ref[...], w1_ref[...], preferred_element_type=jnp.float32)
    d = disb_ref[...]
    out_ref[0] = d * xw[:, :HH]
    out_ref[1] = d * xw[:, HH:]


def _tc1_call(x, W1, disb):
    return pl.pallas_call(
        _tc1_body,
        grid=(GRID,),
        in_specs=[
            pl.BlockSpec((BN, D), lambda i: (i, 0)),
            pl.BlockSpec((D, H), lambda i: (0, 0)),
            pl.BlockSpec((BN, HH), lambda i: (i, 0)),
        ],
        out_specs=pl.BlockSpec((2, BN, HH), lambda i: (0, i, 0)),
        out_shape=jax.ShapeDtypeStruct((2, N, HH), jnp.float32),
    )(x, W1, disb)


def _tc2_body(a_ref, hp_ref, disb_ref, b1_ref, w2_ref, out_ref):
    d = disb_ref[...]
    b1 = b1_ref[...]
    h1l = jnp.maximum(d * (a_ref[0] + hp_ref[0]) + b1[0], 0.0)
    h1r = jnp.maximum(d * (a_ref[1] + hp_ref[1]) + b1[1], 0.0)
    w2 = w2_ref[...]
    g = (jnp.dot(h1l, w2[:HH, :], preferred_element_type=jnp.float32)
         + jnp.dot(h1r, w2[HH:, :], preferred_element_type=jnp.float32))
    out_ref[0] = d * g[:, :HH]
    out_ref[1] = d * g[:, HH:]


def _tc2_call(A1, hp, disb, b1r, W2):
    return pl.pallas_call(
        _tc2_body,
        grid=(GRID,),
        in_specs=[
            pl.BlockSpec((2, BN, HH), lambda i: (0, i, 0)),
            pl.BlockSpec((2, BN, HH), lambda i: (0, i, 0)),
            pl.BlockSpec((BN, HH), lambda i: (i, 0)),
            pl.BlockSpec((2, HH), lambda i: (0, 0)),
            pl.BlockSpec((H, H), lambda i: (0, 0)),
        ],
        out_specs=pl.BlockSpec((2, BN, HH), lambda i: (0, i, 0)),
        out_shape=jax.ShapeDtypeStruct((2, N, HH), jnp.float32),
    )(A1, hp, disb, b1r, W2)


def _tc3_body(a_ref, gp_ref, disb_ref, b2_ref, wf_ref, bf_ref, out_ref):
    d = disb_ref[...]
    b2 = b2_ref[...]
    h2l = jnp.maximum(d * (a_ref[0] + gp_ref[0]) + b2[0], 0.0)
    h2r = jnp.maximum(d * (a_ref[1] + gp_ref[1]) + b2[1], 0.0)
    wf = wf_ref[...]
    out_ref[...] = (jnp.dot(h2l, wf[:HH, :], preferred_element_type=jnp.float32)
                    + jnp.dot(h2r, wf[HH:, :], preferred_element_type=jnp.float32)
                    + bf_ref[...])


def _tc3_call(A2, gp, disb, b2r, Wf, bfr):
    return pl.pallas_call(
        _tc3_body,
        grid=(GRID,),
        in_specs=[
            pl.BlockSpec((2, BN, HH), lambda i: (0, i, 0)),
            pl.BlockSpec((2, BN, HH), lambda i: (0, i, 0)),
            pl.BlockSpec((BN, HH), lambda i: (i, 0)),
            pl.BlockSpec((2, HH), lambda i: (0, 0)),
            pl.BlockSpec((H, O), lambda i: (0, 0)),
            pl.BlockSpec((1, O), lambda i: (0, 0)),
        ],
        out_specs=pl.BlockSpec((BN, O), lambda i: (i, 0)),
        out_shape=jax.ShapeDtypeStruct((N, O), jnp.float32),
    )(A2, gp, disb, b2r, Wf, bfr)


def kernel(x, edge_index, edge_attr, W1, b1, W2, b2, We, be, Wf, bf):
    del edge_attr, We, be
    src = edge_index[0]
    dst = edge_index[1]

    degp = _deg_call(dst)
    disb2d = _dis_call(degp.reshape(NC, NPAD // 128, 128))
    dis = disb2d.reshape(NPAD)[:N]
    disb = jnp.broadcast_to(dis[:, None], (N, HH))

    pad_iota = jnp.arange(EP - E, dtype=jnp.int32)
    srcp = jnp.concatenate([src, pad_iota % N])
    dstp = jnp.concatenate([dst, N + pad_iota % (NPAD - N)])

    hp = _tc1_call(x, W1, disb)
    a1 = _seg_call(hp.reshape(2 * N, HH), srcp,
                   dstp).reshape(2, NPAD, HH)
    gp = _tc2_call(a1, hp, disb, b1.reshape(NC, HH), W2)
    a2 = _seg_call(gp.reshape(2 * N, HH), srcp,
                   dstp).reshape(2, NPAD, HH)
    return _tc3_call(a2, gp, disb, b2.reshape(NC, HH), Wf, bf.reshape(1, O))


# speedup vs baseline: 2.7345x; 1.0266x over previous
import functools

import jax
import jax.numpy as jnp
from jax import lax
from jax.experimental import pallas as pl
from jax.experimental.pallas import tpu as pltpu
from jax.experimental.pallas import tpu_sc as plsc

N = 10000
E = 320000
D = 128
H = 256
O = 128
HH = H // 2

NC = 2
NS = 16
LANES = 16

NPAD = 10240
RPT = NPAD // NS

SEG_C = 80
SEG_IBUF = 8
SEG_RBUF = 4
EPT = 20480
EP = EPT * NS
SEG_NCHUNK = EPT // SEG_C

DEG_E = E // (NC * NS)
DEG_C = 80
DEG_NCHUNK = DEG_E // DEG_C
DEG_NBUF = 5
DEG_NOUTER = DEG_NCHUNK // DEG_NBUF

_mesh = plsc.VectorSubcoreMesh(core_axis_name="c", subcore_axis_name="s")


def _zero_vmem_1d(ref, nwords):
    def body(i, _):
        ref[pl.ds(i * LANES, LANES)] = jnp.zeros((LANES,), jnp.float32)
        return 0
    lax.fori_loop(0, nwords // LANES, body, 0)


def _deg_body(dst_hbm, out_hbm, deg_sh,
              db0, db1, db2, db3, db4, ones, rbuf,
              ds0, ds1, ds2, ds3, ds4):
    c = lax.axis_index("c")
    s = lax.axis_index("s")
    tid = c * NS + s
    dibs = (db0, db1, db2, db3, db4)
    sems = (ds0, ds1, ds2, ds3, ds4)

    seg = NPAD // NS
    r0 = s * seg
    _zero_vmem_1d(rbuf, seg)
    pltpu.sync_copy(rbuf, deg_sh.at[pl.ds(r0, seg)])

    def fill_ones(i, _):
        ones[pl.ds(i * LANES, LANES)] = jnp.ones((LANES,), jnp.float32)
        return 0
    lax.fori_loop(0, DEG_C // LANES, fill_ones, 0)
    plsc.subcore_barrier()

    base = tid * DEG_E

    def load_idx(k, b):
        pltpu.async_copy(dst_hbm.at[pl.ds(base + k * DEG_C, DEG_C)],
                         dibs[b], sems[b])

    for b in range(DEG_NBUF):
        load_idx(b, b)

    def outer(ko, _):
        for b in range(DEG_NBUF):
            k = ko * DEG_NBUF + b
            pltpu.make_async_copy(
                dst_hbm.at[pl.ds(base + k * DEG_C, DEG_C)], dibs[b], sems[b]
            ).wait()
            pltpu.sync_copy(ones, deg_sh.at[dibs[b]], add=True)
            kp = k + DEG_NBUF

            @pl.when(kp < DEG_NCHUNK)
            def _():
                load_idx(kp, b)
        return 0

    lax.fori_loop(0, DEG_NOUTER, outer, 0)
    plsc.subcore_barrier()
    pltpu.sync_copy(deg_sh.at[pl.ds(r0, seg)], out_hbm.at[c, pl.ds(r0, seg)])


_deg_call = pl.kernel(
    _deg_body,
    out_type=jax.ShapeDtypeStruct((NC, NPAD), jnp.float32),
    mesh=_mesh,
    scratch_types=(
        [pltpu.VMEM_SHARED((NPAD,), jnp.float32)]
        + [pltpu.VMEM((DEG_C,), jnp.int32) for _ in range(DEG_NBUF)]
        + [pltpu.VMEM((DEG_C,), jnp.float32),
           pltpu.VMEM((NPAD // NS,), jnp.float32)]
        + [pltpu.SemaphoreType.DMA for _ in range(DEG_NBUF)]
    ),
)


def _seg_body(hp_hbm, src_hbm, dst_hbm, out_hbm, acc,
              sib0, sib1, sib2, sib3, sib4, sib5, sib6, sib7,
              dib0, dib1, dib2, dib3, dib4, dib5, dib6, dib7,
              rb0, rb1, rb2, rb3,
              gs0, gs1, gs2, gs3, ss0, ss1, ss2, ss3,
              is0, is1, is2, is3, is4, is5, is6, is7):
    c = lax.axis_index("c")
    s = lax.axis_index("s")
    sibs = (sib0, sib1, sib2, sib3, sib4, sib5, sib6, sib7)
    dibs = (dib0, dib1, dib2, dib3, dib4, dib5, dib6, dib7)
    rbs = (rb0, rb1, rb2, rb3)
    gsems = (gs0, gs1, gs2, gs3)
    ssems = (ss0, ss1, ss2, ss3)
    isems = (is0, is1, is2, is3, is4, is5, is6, is7)

    def zrb(i, _):
        rb0[i // (HH // LANES),
            pl.ds((i % (HH // LANES)) * LANES, LANES)] = (
                jnp.zeros((LANES,), jnp.float32))
        return 0
    lax.fori_loop(0, SEG_C * HH // LANES, zrb, 0)
    r0 = s * RPT
    for j in range(RPT // SEG_C):
        pltpu.sync_copy(rb0, acc.at[pl.ds(r0 + j * SEG_C, SEG_C)])
    plsc.subcore_barrier()

    base = s * EPT
    off = c * N

    def idx_load(q, p):
        pltpu.async_copy(src_hbm.at[pl.ds(base + q * SEG_C, SEG_C)],
                         sibs[p], isems[p])
        pltpu.async_copy(dst_hbm.at[pl.ds(base + q * SEG_C, SEG_C)],
                         dibs[p], isems[p])

    def idx_wait(p):
        pltpu.make_async_copy(src_hbm.at[pl.ds(0, SEG_C)], sibs[p],
                              isems[p]).wait()
        pltpu.make_async_copy(dst_hbm.at[pl.ds(0, SEG_C)], dibs[p],
                              isems[p]).wait()

    def rebase_and_gather(p, b):
        for i in range(SEG_C // LANES):
            sl = pl.ds(i * LANES, LANES)
            sibs[p][sl] = sibs[p][sl] + off
        pltpu.async_copy(hp_hbm.at[sibs[p]], rbs[b], gsems[b])

    def gather_wait(p, b):
        pltpu.make_async_copy(hp_hbm.at[sibs[p]], rbs[b], gsems[b]).wait()

    def scatter_start(p, b):
        pltpu.async_copy(rbs[b], acc.at[dibs[p]], ssems[b], add=True)

    def scatter_wait(p, b):
        pltpu.make_async_copy(rbs[b], acc.at[dibs[p]], ssems[b]).wait()

    for q in range(6):
        idx_load(q, q)
    for q in range(2):
        idx_wait(q)
        rebase_and_gather(q, q)

    def outer(ko, _):
        for u in range(SEG_IBUF):
            k = ko * SEG_IBUF + u
            b = u % SEG_RBUF
            gather_wait(u, b)
            scatter_start(u, b)

            p2 = (u + 6) % SEG_IBUF
            b2 = (u + 2) % SEG_RBUF
            if u < 2:
                @pl.when(k >= 2)
                def _():
                    scatter_wait(p2, b2)
            else:
                scatter_wait(p2, b2)

            kf = k + 6

            @pl.when(kf < SEG_NCHUNK)
            def _():
                idx_load(kf, p2)

            kp = k + 2
            up = (u + 2) % SEG_IBUF

            @pl.when(kp < SEG_NCHUNK)
            def _():
                idx_wait(up)
                rebase_and_gather(up, b2)
        return 0

    lax.fori_loop(0, SEG_NCHUNK // SEG_IBUF, outer, 0)
    scatter_wait(6, 2)
    scatter_wait(7, 3)
    plsc.subcore_barrier()
    pltpu.sync_copy(acc.at[pl.ds(r0, RPT)],
                    out_hbm.at[pl.ds(c * NPAD + r0, RPT)])


_seg_call = pl.kernel(
    _seg_body,
    out_type=jax.ShapeDtypeStruct((2 * NPAD, HH), jnp.float32),
    mesh=_mesh,
    scratch_types=(
        [pltpu.VMEM_SHARED((NPAD, HH), jnp.float32)]
        + [pltpu.VMEM((SEG_C,), jnp.int32) for _ in range(2 * SEG_IBUF)]
        + [pltpu.VMEM((SEG_C, HH), jnp.float32) for _ in range(SEG_RBUF)]
        + [pltpu.SemaphoreType.DMA for _ in range(2 * SEG_RBUF + SEG_IBUF)]
    ),
)


BN = 1000
GRID = N // BN


def _dis_body(degp_ref, out_ref):
    deg = degp_ref[0] + degp_ref[1] + 1.0
    out_ref[...] = lax.rsqrt(deg)


def _dis_call(degp3):
    return pl.pallas_call(
        _dis_body,
        out_shape=jax.ShapeDtypeStruct((NPAD // 128, 128), jnp.float32),
    )(degp3)


def _tc1_body(x_ref, w1_ref, disb_ref, out_ref):
    xw = jnp.dot(x_ref[...], w1_ref[...], preferred_element_type=jnp.float32)
    d = disb_ref[...]
    out_ref[0] = d * xw[:, :HH]
    out_ref[1] = d * xw[:, HH:]


def _tc1_call(x, W1, disb):
    return pl.pallas_call(
        _tc1_body,
        grid=(GRID,),
        in_specs=[
            pl.BlockSpec((BN, D), lambda i: (i, 0)),
            pl.BlockSpec((D, H), lambda i: (0, 0)),
            pl.BlockSpec((BN, HH), lambda i: (i, 0)),
        ],
        out_specs=pl.BlockSpec((2, BN, HH), lambda i: (0, i, 0)),
        out_shape=jax.ShapeDtypeStruct((2, N, HH), jnp.float32),
    )(x, W1, disb)


def _tc2_body(a_ref, hp_ref, disb_ref, b1_ref, w2_ref, out_ref):
    d = disb_ref[...]
    b1 = b1_ref[...]
    h1l = jnp.maximum(d * (a_ref[0] + hp_ref[0]) + b1[0], 0.0)
    h1r = jnp.maximum(d * (a_ref[1] + hp_ref[1]) + b1[1], 0.0)
    w2 = w2_ref[...]
    g = (jnp.dot(h1l, w2[:HH, :], preferred_element_type=jnp.float32)
         + jnp.dot(h1r, w2[HH:, :], preferred_element_type=jnp.float32))
    out_ref[0] = d * g[:, :HH]
    out_ref[1] = d * g[:, HH:]


def _tc2_call(A1, hp, disb, b1r, W2):
    return pl.pallas_call(
        _tc2_body,
        grid=(GRID,),
        in_specs=[
            pl.BlockSpec((2, BN, HH), lambda i: (0, i, 0)),
            pl.BlockSpec((2, BN, HH), lambda i: (0, i, 0)),
            pl.BlockSpec((BN, HH), lambda i: (i, 0)),
            pl.BlockSpec((2, HH), lambda i: (0, 0)),
            pl.BlockSpec((H, H), lambda i: (0, 0)),
        ],
        out_specs=pl.BlockSpec((2, BN, HH), lambda i: (0, i, 0)),
        out_shape=jax.ShapeDtypeStruct((2, N, HH), jnp.float32),
    )(A1, hp, disb, b1r, W2)


def _tc3_body(a_ref, gp_ref, disb_ref, b2_ref, wf_ref, bf_ref, out_ref):
    d = disb_ref[...]
    b2 = b2_ref[...]
    h2l = jnp.maximum(d * (a_ref[0] + gp_ref[0]) + b2[0], 0.0)
    h2r = jnp.maximum(d * (a_ref[1] + gp_ref[1]) + b2[1], 0.0)
    wf = wf_ref[...]
    out_ref[...] = (jnp.dot(h2l, wf[:HH, :], preferred_element_type=jnp.float32)
                    + jnp.dot(h2r, wf[HH:, :], preferred_element_type=jnp.float32)
                    + bf_ref[...])


def _tc3_call(A2, gp, disb, b2r, Wf, bfr):
    return pl.pallas_call(
        _tc3_body,
        grid=(GRID,),
        in_specs=[
            pl.BlockSpec((2, BN, HH), lambda i: (0, i, 0)),
            pl.BlockSpec((2, BN, HH), lambda i: (0, i, 0)),
            pl.BlockSpec((BN, HH), lambda i: (i, 0)),
            pl.BlockSpec((2, HH), lambda i: (0, 0)),
            pl.BlockSpec((H, O), lambda i: (0, 0)),
            pl.BlockSpec((1, O), lambda i: (0, 0)),
        ],
        out_specs=pl.BlockSpec((BN, O), lambda i: (i, 0)),
        out_shape=jax.ShapeDtypeStruct((N, O), jnp.float32),
    )(A2, gp, disb, b2r, Wf, bfr)


def kernel(x, edge_index, edge_attr, W1, b1, W2, b2, We, be, Wf, bf):
    del edge_attr, We, be
    src = edge_index[0]
    dst = edge_index[1]

    degp = _deg_call(dst)
    disb2d = _dis_call(degp.reshape(NC, NPAD // 128, 128))
    dis = disb2d.reshape(NPAD)[:N]
    disb = jnp.broadcast_to(dis[:, None], (N, HH))

    pad_iota = jnp.arange(EP - E, dtype=jnp.int32)
    srcp = jnp.concatenate([src, pad_iota % N])
    dstp = jnp.concatenate([dst, N + pad_iota % (NPAD - N)])

    hp = _tc1_call(x, W1, disb)
    a1 = _seg_call(hp.reshape(2 * N, HH), srcp,
                   dstp).reshape(2, NPAD, HH)
    gp = _tc2_call(a1, hp, disb, b1.reshape(NC, HH), W2)
    a2 = _seg_call(gp.reshape(2 * N, HH), srcp,
                   dstp).reshape(2, NPAD, HH)
    return _tc3_call(a2, gp, disb, b2.reshape(NC, HH), Wf, bf.reshape(1, O))


# speedup vs baseline: 2.9261x; 1.0701x over previous
import functools

import jax
import jax.numpy as jnp
from jax import lax
from jax.experimental import pallas as pl
from jax.experimental.pallas import tpu as pltpu
from jax.experimental.pallas import tpu_sc as plsc

N = 10000
E = 320000
D = 128
H = 256
O = 128
HH = H // 2

NC = 2
NS = 16
LANES = 16

NPAD = 10240
RPT = NPAD // NS

SEG_C = 128
SEG_IBUF = 4
EPT = 20480
EP = EPT * NS
SEG_NCHUNK = EPT // SEG_C

DEG_E = E // (NC * NS)
DEG_C = 80
DEG_NCHUNK = DEG_E // DEG_C
DEG_NBUF = 5
DEG_NOUTER = DEG_NCHUNK // DEG_NBUF

_mesh = plsc.VectorSubcoreMesh(core_axis_name="c", subcore_axis_name="s")


def _zero_vmem_1d(ref, nwords):
    def body(i, _):
        ref[pl.ds(i * LANES, LANES)] = jnp.zeros((LANES,), jnp.float32)
        return 0
    lax.fori_loop(0, nwords // LANES, body, 0)


def _deg_body(dst_hbm, out_hbm, deg_sh,
              db0, db1, db2, db3, db4, ones, rbuf,
              ds0, ds1, ds2, ds3, ds4):
    c = lax.axis_index("c")
    s = lax.axis_index("s")
    tid = c * NS + s
    dibs = (db0, db1, db2, db3, db4)
    sems = (ds0, ds1, ds2, ds3, ds4)

    seg = NPAD // NS
    r0 = s * seg
    _zero_vmem_1d(rbuf, seg)
    pltpu.sync_copy(rbuf, deg_sh.at[pl.ds(r0, seg)])

    def fill_ones(i, _):
        ones[pl.ds(i * LANES, LANES)] = jnp.ones((LANES,), jnp.float32)
        return 0
    lax.fori_loop(0, DEG_C // LANES, fill_ones, 0)
    plsc.subcore_barrier()

    base = tid * DEG_E

    def load_idx(k, b):
        pltpu.async_copy(dst_hbm.at[pl.ds(base + k * DEG_C, DEG_C)],
                         dibs[b], sems[b])

    for b in range(DEG_NBUF):
        load_idx(b, b)

    def outer(ko, _):
        for b in range(DEG_NBUF):
            k = ko * DEG_NBUF + b
            pltpu.make_async_copy(
                dst_hbm.at[pl.ds(base + k * DEG_C, DEG_C)], dibs[b], sems[b]
            ).wait()
            pltpu.sync_copy(ones, deg_sh.at[dibs[b]], add=True)
            kp = k + DEG_NBUF

            @pl.when(kp < DEG_NCHUNK)
            def _():
                load_idx(kp, b)
        return 0

    lax.fori_loop(0, DEG_NOUTER, outer, 0)
    plsc.subcore_barrier()
    pltpu.sync_copy(deg_sh.at[pl.ds(r0, seg)], out_hbm.at[c, pl.ds(r0, seg)])


_deg_call = pl.kernel(
    _deg_body,
    out_type=jax.ShapeDtypeStruct((NC, NPAD), jnp.float32),
    mesh=_mesh,
    scratch_types=(
        [pltpu.VMEM_SHARED((NPAD,), jnp.float32)]
        + [pltpu.VMEM((DEG_C,), jnp.int32) for _ in range(DEG_NBUF)]
        + [pltpu.VMEM((DEG_C,), jnp.float32),
           pltpu.VMEM((NPAD // NS,), jnp.float32)]
        + [pltpu.SemaphoreType.DMA for _ in range(DEG_NBUF)]
    ),
)


def _seg_body(hp_hbm, src_hbm, dst_hbm, out_hbm, acc,
              sib0, sib1, sib2, sib3, dib0, dib1, dib2, dib3,
              rb0, rb1, gs0, gs1, is0, is1, is2, is3):
    c = lax.axis_index("c")
    s = lax.axis_index("s")
    sibs = (sib0, sib1, sib2, sib3)
    dibs = (dib0, dib1, dib2, dib3)
    rbs = (rb0, rb1)
    gsems = (gs0, gs1)
    isems = (is0, is1, is2, is3)

    def zrb(i, _):
        rb0[i // (HH // LANES),
            pl.ds((i % (HH // LANES)) * LANES, LANES)] = (
                jnp.zeros((LANES,), jnp.float32))
        return 0
    lax.fori_loop(0, SEG_C * HH // LANES, zrb, 0)
    r0 = s * RPT
    for j in range(RPT // SEG_C):
        pltpu.sync_copy(rb0, acc.at[pl.ds(r0 + j * SEG_C, SEG_C)])
    plsc.subcore_barrier()

    base = s * EPT
    off = c * N

    def idx_load(q, p):
        pltpu.async_copy(src_hbm.at[pl.ds(base + q * SEG_C, SEG_C)],
                         sibs[p], isems[p])
        pltpu.async_copy(dst_hbm.at[pl.ds(base + q * SEG_C, SEG_C)],
                         dibs[p], isems[p])

    def idx_wait(p):
        pltpu.make_async_copy(src_hbm.at[pl.ds(0, SEG_C)], sibs[p],
                              isems[p]).wait()
        pltpu.make_async_copy(dst_hbm.at[pl.ds(0, SEG_C)], dibs[p],
                              isems[p]).wait()

    def rebase_and_gather(p, b):
        for i in range(SEG_C // LANES):
            sl = pl.ds(i * LANES, LANES)
            sibs[p][sl] = sibs[p][sl] + off
        pltpu.async_copy(hp_hbm.at[sibs[p]], rbs[b], gsems[b])

    for q in range(SEG_IBUF):
        idx_load(q, q)
    for q in range(2):
        idx_wait(q)
        rebase_and_gather(q, q)

    def outer(ko, _):
        for u in range(SEG_IBUF):
            k = ko * SEG_IBUF + u
            b = u % 2
            pltpu.make_async_copy(hp_hbm.at[sibs[u]], rbs[b],
                                  gsems[b]).wait()
            pltpu.sync_copy(rbs[b], acc.at[dibs[u]], add=True)

            kf = k + SEG_IBUF

            @pl.when(kf < SEG_NCHUNK)
            def _():
                idx_load(kf, u)

            kp = k + 2
            up = (u + 2) % SEG_IBUF

            @pl.when(kp < SEG_NCHUNK)
            def _():
                idx_wait(up)
                rebase_and_gather(up, b)
        return 0

    lax.fori_loop(0, SEG_NCHUNK // SEG_IBUF, outer, 0)
    plsc.subcore_barrier()
    pltpu.sync_copy(acc.at[pl.ds(r0, RPT)],
                    out_hbm.at[pl.ds(c * NPAD + r0, RPT)])


_seg_call = pl.kernel(
    _seg_body,
    out_type=jax.ShapeDtypeStruct((2 * NPAD, HH), jnp.float32),
    mesh=_mesh,
    scratch_types=(
        [pltpu.VMEM_SHARED((NPAD, HH), jnp.float32)]
        + [pltpu.VMEM((SEG_C,), jnp.int32) for _ in range(2 * SEG_IBUF)]
        + [pltpu.VMEM((SEG_C, HH), jnp.float32) for _ in range(2)]
        + [pltpu.SemaphoreType.DMA for _ in range(2 + SEG_IBUF)]
    ),
)


BN = 1000
GRID = N // BN


def _dis_body(degp_ref, out_ref):
    deg = degp_ref[0] + degp_ref[1] + 1.0
    out_ref[...] = lax.rsqrt(deg)


def _dis_call(degp3):
    return pl.pallas_call(
        _dis_body,
        out_shape=jax.ShapeDtypeStruct((NPAD // 128, 128), jnp.float32),
    )(degp3)


def _tc1_body(x_ref, w1_ref, disb_ref, out_ref):
    xw = jnp.dot(x_ref[...], w1_ref[...], preferred_element_type=jnp.float32)
    d = disb_ref[...]
    out_ref[0] = d * xw[:, :HH]
    out_ref[1] = d * xw[:, HH:]


def _tc1_call(x, W1, disb):
    return pl.pallas_call(
        _tc1_body,
        grid=(GRID,),
        in_specs=[
            pl.BlockSpec((BN, D), lambda i: (i, 0)),
            pl.BlockSpec((D, H), lambda i: (0, 0)),
            pl.BlockSpec((BN, HH), lambda i: (i, 0)),
        ],
        out_specs=pl.BlockSpec((2, BN, HH), lambda i: (0, i, 0)),
        out_shape=jax.ShapeDtypeStruct((2, N, HH), jnp.float32),
    )(x, W1, disb)


def _tc2_body(a_ref, hp_ref, disb_ref, b1_ref, w2_ref, out_ref):
    d = disb_ref[...]
    b1 = b1_ref[...]
    h1l = jnp.maximum(d * (a_ref[0] + hp_ref[0]) + b1[0], 0.0)
    h1r = jnp.maximum(d * (a_ref[1] + hp_ref[1]) + b1[1], 0.0)
    w2 = w2_ref[...]
    g = (jnp.dot(h1l, w2[:HH, :], preferred_element_type=jnp.float32)
         + jnp.dot(h1r, w2[HH:, :], preferred_element_type=jnp.float32))
    out_ref[0] = d * g[:, :HH]
    out_ref[1] = d * g[:, HH:]


def _tc2_call(A1, hp, disb, b1r, W2):
    return pl.pallas_call(
        _tc2_body,
        grid=(GRID,),
        in_specs=[
            pl.BlockSpec((2, BN, HH), lambda i: (0, i, 0)),
            pl.BlockSpec((2, BN, HH), lambda i: (0, i, 0)),
            pl.BlockSpec((BN, HH), lambda i: (i, 0)),
            pl.BlockSpec((2, HH), lambda i: (0, 0)),
            pl.BlockSpec((H, H), lambda i: (0, 0)),
        ],
        out_specs=pl.BlockSpec((2, BN, HH), lambda i: (0, i, 0)),
        out_shape=jax.ShapeDtypeStruct((2, N, HH), jnp.float32),
    )(A1, hp, disb, b1r, W2)


def _tc3_body(a_ref, gp_ref, disb_ref, b2_ref, wf_ref, bf_ref, out_ref):
    d = disb_ref[...]
    b2 = b2_ref[...]
    h2l = jnp.maximum(d * (a_ref[0] + gp_ref[0]) + b2[0], 0.0)
    h2r = jnp.maximum(d * (a_ref[1] + gp_ref[1]) + b2[1], 0.0)
    wf = wf_ref[...]
    out_ref[...] = (jnp.dot(h2l, wf[:HH, :], preferred_element_type=jnp.float32)
                    + jnp.dot(h2r, wf[HH:, :], preferred_element_type=jnp.float32)
                    + bf_ref[...])


def _tc3_call(A2, gp, disb, b2r, Wf, bfr):
    return pl.pallas_call(
        _tc3_body,
        grid=(GRID,),
        in_specs=[
            pl.BlockSpec((2, BN, HH), lambda i: (0, i, 0)),
            pl.BlockSpec((2, BN, HH), lambda i: (0, i, 0)),
            pl.BlockSpec((BN, HH), lambda i: (i, 0)),
            pl.BlockSpec((2, HH), lambda i: (0, 0)),
            pl.BlockSpec((H, O), lambda i: (0, 0)),
            pl.BlockSpec((1, O), lambda i: (0, 0)),
        ],
        out_specs=pl.BlockSpec((BN, O), lambda i: (i, 0)),
        out_shape=jax.ShapeDtypeStruct((N, O), jnp.float32),
    )(A2, gp, disb, b2r, Wf, bfr)


def kernel(x, edge_index, edge_attr, W1, b1, W2, b2, We, be, Wf, bf):
    del edge_attr, We, be
    src = edge_index[0]
    dst = edge_index[1]

    degp = _deg_call(dst)
    disb2d = _dis_call(degp.reshape(NC, NPAD // 128, 128))
    dis = disb2d.reshape(NPAD)[:N]
    disb = jnp.broadcast_to(dis[:, None], (N, HH))

    pad_iota = jnp.arange(EP - E, dtype=jnp.int32)
    srcp = jnp.concatenate([src, pad_iota % N])
    dstp = jnp.concatenate([dst, N + pad_iota % (NPAD - N)])

    hp = _tc1_call(x, W1, disb)
    a1 = _seg_call(hp.reshape(2 * N, HH), srcp,
                   dstp).reshape(2, NPAD, HH)
    gp = _tc2_call(a1, hp, disb, b1.reshape(NC, HH), W2)
    a2 = _seg_call(gp.reshape(2 * N, HH), srcp,
                   dstp).reshape(2, NPAD, HH)
    return _tc3_call(a2, gp, disb, b2.reshape(NC, HH), Wf, bf.reshape(1, O))
